# baseline (speedup 1.0000x reference)
import functools

import jax
import jax.numpy as jnp
from jax.experimental import pallas as pl
from jax.experimental.pallas import tpu as pltpu

SOS = 1
EOS = 2


def _proj_body(h_ref, wout_ref, bout_ref, lg_ref, amax_ref, w16_ref,
               rmax_s, ridx_s, *, B, V, TILE, J, cast):
    j = pl.program_id(0)

    @pl.when(j == 0)
    def _i():
        rmax_s[...] = jnp.full((B, 1), -jnp.inf, jnp.float32)
        ridx_s[...] = jnp.zeros((B, 1), jnp.int32)

    hb = h_ref[...].astype(jnp.bfloat16)
    if cast == "dump":
        wb = wout_ref[...].astype(jnp.bfloat16)
        w16_ref[...] = wb
    elif cast == "f32":
        wb = wout_ref[...].astype(jnp.bfloat16)
    else:
        wb = wout_ref[...]
    logits = (jnp.dot(hb, wb, preferred_element_type=jnp.float32)
              + bout_ref[...])
    lg_ref[...] = logits
    colv = jax.lax.broadcasted_iota(jnp.int32, (B, TILE), 1) + j * TILE
    lm = jnp.where(colv < V, logits, -jnp.inf)
    tmax = jnp.max(lm, axis=1, keepdims=True)
    targ = jnp.min(jnp.where(lm == tmax, colv, V), axis=1, keepdims=True)
    upd = tmax > rmax_s[...]
    ridx_s[...] = jnp.where(upd, targ, ridx_s[...])
    rmax_s[...] = jnp.where(upd, tmax, rmax_s[...])

    @pl.when(j == J - 1)
    def _f():
        amax_ref[...] = ridx_s[...]


def _project(h, W, b_out, cast):
    B = h.shape[0]
    D, V = W.shape
    TILE = 8192
    J = pl.cdiv(V, TILE)
    body = functools.partial(_proj_body, B=B, V=V, TILE=TILE, J=J, cast=cast)
    out_shape = [
        jax.ShapeDtypeStruct((B, V), jnp.float32),
        jax.ShapeDtypeStruct((B, 1), jnp.int32),
        jax.ShapeDtypeStruct((D, V), jnp.bfloat16),
    ]
    out_specs = [
        pl.BlockSpec((B, TILE), lambda j: (0, j)),
        pl.BlockSpec((B, 1), lambda j: (0, 0)),
        pl.BlockSpec((D, TILE), lambda j: (0, j)),
    ]
    if cast != "dump":
        out_shape = out_shape[:2]
        out_specs = out_specs[:2]
    outs = pl.pallas_call(
        body if cast == "dump" else (lambda *a: body(*a[:5], None, *a[5:])),
        grid=(J,),
        in_specs=[
            pl.BlockSpec((B, D), lambda j: (0, 0)),
            pl.BlockSpec((D, TILE), lambda j: (0, j)),
            pl.BlockSpec((1, TILE), lambda j: (0, j)),
        ],
        out_specs=out_specs,
        out_shape=out_shape,
        scratch_shapes=[
            pltpu.VMEM((B, 1), jnp.float32),
            pltpu.VMEM((B, 1), jnp.int32),
        ],
        compiler_params=pltpu.CompilerParams(
            dimension_semantics=("arbitrary",)),
    )(h, W, b_out.reshape(1, V))
    if cast == "dump":
        logits, amax, w16 = outs
        return logits, amax[:, 0], w16
    logits, amax = outs
    return logits, amax[:, 0], None


def _gather_body(idx_sm, embt_ref, out_ref, sem):
    B = out_ref.shape[0]
    for i in range(B):
        pltpu.make_async_copy(embt_ref.at[pl.ds(idx_sm[i], 1), :],
                              out_ref.at[pl.ds(i, 1), :], sem).start()
    for i in range(B):
        pltpu.make_async_copy(embt_ref.at[pl.ds(0, 1), :],
                              out_ref.at[pl.ds(i, 1), :], sem).wait()


def _gather(emb_table, idx):
    B = idx.shape[0]
    D = emb_table.shape[1]
    return pl.pallas_call(
        _gather_body,
        grid_spec=pltpu.PrefetchScalarGridSpec(
            num_scalar_prefetch=1,
            grid=(1,),
            in_specs=[pl.BlockSpec(memory_space=pl.ANY)],
            out_specs=pl.BlockSpec((B, D), lambda i, *_: (0, 0)),
            scratch_shapes=[pltpu.SemaphoreType.DMA],
        ),
        out_shape=jax.ShapeDtypeStruct((B, D), jnp.float32),
    )(idx, emb_table)


def kernel(semantics, styles, UNK_embeds, UNK_lengths, UNK_word_ids, emb_table,
           W_x, W_h, b_h, W_out, b_out, W_att, w_gate, max_generation_steps):
    B = semantics.shape[0]
    U = UNK_embeds.shape[1]
    D = W_h.shape[0]
    last_output = jnp.full((B,), SOS, dtype=jnp.int32)
    last_UNK_embeds = jnp.zeros((B, D), jnp.float32)
    h = jnp.zeros((B, D), jnp.float32)
    UNK_mask = jnp.ones((B, U), jnp.float32)
    idx_range = jnp.arange(U)
    len_mask = (idx_range[None, :] < UNK_lengths[:, None]).astype(jnp.float32)
    all_outputs, all_UNK_weights, all_preds = [], [], []
    W16 = None
    for t in range(4):
        word_emb = _gather(emb_table, last_output)
        x = jnp.concatenate([semantics, styles, word_emb, last_UNK_embeds], axis=-1)
        h = jnp.tanh(x @ W_x + h @ W_h + b_h)
        logits, raw_argmax, _ = _project(h, W_out, b_out, cast="f32")
        att = jnp.einsum('bd,bud->bu', h @ W_att, UNK_embeds) / jnp.sqrt(float(D))
        valid = (UNK_mask > 0).astype(jnp.float32) * len_mask
        att = jnp.where(valid > 0, att, -1e9)
        gate = jnp.sum(h * w_gate, axis=-1, keepdims=True)
        UNK_weights = jax.nn.softmax(jnp.concatenate([gate, att], axis=-1), axis=-1)
        last_UNK_embeds = jnp.einsum('bu,bud->bd', UNK_weights[:, 1:], UNK_embeds)
        word_preds = raw_argmax.astype(jnp.int32)
        UNK_argmax = jnp.argmax(UNK_weights[:, 1:], axis=-1)
        is_UNK = (UNK_weights[:, 0] < 0.5).astype(jnp.int32)
        UNK_step_indices = jnp.take_along_axis(UNK_word_ids, UNK_argmax[:, None], axis=1)[:, 0].astype(jnp.int32)
        word_preds = word_preds * (1 - is_UNK) + UNK_step_indices * is_UNK
        last_output = word_preds
        UNK_mask = UNK_mask - (is_UNK[:, None] * (UNK_argmax[:, None] == idx_range[None, :]).astype(jnp.int32)).astype(jnp.float32)
        all_outputs.append(logits)
        all_UNK_weights.append(UNK_weights)
        all_preds.append(word_preds)
    return (jnp.stack(all_outputs, axis=1),
            jnp.stack(all_UNK_weights, axis=1),
            jnp.stack(all_preds, axis=1))


# speedup vs baseline: 1.1082x; 1.1082x over previous
import functools

import jax
import jax.numpy as jnp
from jax.experimental import pallas as pl
from jax.experimental.pallas import tpu as pltpu

SOS = 1
EOS = 2


def _proj_body(h_ref, wout_ref, bout_ref, lg_ref, amax_ref, w16_ref,
               rmax_s, ridx_s, *, B, V, TILE, J, cast):
    j = pl.program_id(0)

    @pl.when(j == 0)
    def _i():
        rmax_s[...] = jnp.full((B, 1), -jnp.inf, jnp.float32)
        ridx_s[...] = jnp.zeros((B, 1), jnp.int32)

    hb = h_ref[...].astype(jnp.bfloat16)
    if cast == "dump":
        wb = wout_ref[...].astype(jnp.bfloat16)
        w16_ref[...] = wb
    elif cast == "f32":
        wb = wout_ref[...].astype(jnp.bfloat16)
    else:
        wb = wout_ref[...]
    logits = (jnp.dot(hb, wb, preferred_element_type=jnp.float32)
              + bout_ref[...])
    lg_ref[...] = logits
    colv = jax.lax.broadcasted_iota(jnp.int32, (B, TILE), 1) + j * TILE
    lm = jnp.where(colv < V, logits, -jnp.inf)
    tmax = jnp.max(lm, axis=1, keepdims=True)
    targ = jnp.min(jnp.where(lm == tmax, colv, V), axis=1, keepdims=True)
    upd = tmax > rmax_s[...]
    ridx_s[...] = jnp.where(upd, targ, ridx_s[...])
    rmax_s[...] = jnp.where(upd, tmax, rmax_s[...])

    @pl.when(j == J - 1)
    def _f():
        amax_ref[...] = ridx_s[...]


def _project(h, W, b_out, cast):
    B = h.shape[0]
    D, V = W.shape
    TILE = 4096 if cast == "dump" else 16384
    J = pl.cdiv(V, TILE)
    body = functools.partial(_proj_body, B=B, V=V, TILE=TILE, J=J, cast=cast)
    out_shape = [
        jax.ShapeDtypeStruct((B, V), jnp.float32),
        jax.ShapeDtypeStruct((B, 1), jnp.int32),
        jax.ShapeDtypeStruct((D, V), jnp.bfloat16),
    ]
    out_specs = [
        pl.BlockSpec((B, TILE), lambda j: (0, j)),
        pl.BlockSpec((B, 1), lambda j: (0, 0)),
        pl.BlockSpec((D, TILE), lambda j: (0, j)),
    ]
    if cast != "dump":
        out_shape = out_shape[:2]
        out_specs = out_specs[:2]
    outs = pl.pallas_call(
        body if cast == "dump" else (lambda *a: body(*a[:5], None, *a[5:])),
        grid=(J,),
        in_specs=[
            pl.BlockSpec((B, D), lambda j: (0, 0)),
            pl.BlockSpec((D, TILE), lambda j: (0, j)),
            pl.BlockSpec((1, TILE), lambda j: (0, j)),
        ],
        out_specs=out_specs,
        out_shape=out_shape,
        scratch_shapes=[
            pltpu.VMEM((B, 1), jnp.float32),
            pltpu.VMEM((B, 1), jnp.int32),
        ],
        compiler_params=pltpu.CompilerParams(
            dimension_semantics=("arbitrary",)),
    )(h, W, b_out.reshape(1, V))
    if cast == "dump":
        logits, amax, w16 = outs
        return logits, amax[:, 0], w16
    logits, amax = outs
    return logits, amax[:, 0], None


def _gather_body(idx_sm, embt_ref, out_ref, sem):
    B = out_ref.shape[0]
    for i in range(B):
        pltpu.make_async_copy(embt_ref.at[pl.ds(idx_sm[i], 1), :],
                              out_ref.at[pl.ds(i, 1), :], sem).start()
    for i in range(B):
        pltpu.make_async_copy(embt_ref.at[pl.ds(0, 1), :],
                              out_ref.at[pl.ds(i, 1), :], sem).wait()


def _gather(emb_table, idx):
    B = idx.shape[0]
    D = emb_table.shape[1]
    return pl.pallas_call(
        _gather_body,
        grid_spec=pltpu.PrefetchScalarGridSpec(
            num_scalar_prefetch=1,
            grid=(1,),
            in_specs=[pl.BlockSpec(memory_space=pl.ANY)],
            out_specs=pl.BlockSpec((B, D), lambda i, *_: (0, 0)),
            scratch_shapes=[pltpu.SemaphoreType.DMA],
        ),
        out_shape=jax.ShapeDtypeStruct((B, D), jnp.float32),
    )(idx, emb_table)


def kernel(semantics, styles, UNK_embeds, UNK_lengths, UNK_word_ids, emb_table,
           W_x, W_h, b_h, W_out, b_out, W_att, w_gate, max_generation_steps):
    B = semantics.shape[0]
    U = UNK_embeds.shape[1]
    D = W_h.shape[0]
    last_output = jnp.full((B,), SOS, dtype=jnp.int32)
    last_UNK_embeds = jnp.zeros((B, D), jnp.float32)
    h = jnp.zeros((B, D), jnp.float32)
    UNK_mask = jnp.ones((B, U), jnp.float32)
    idx_range = jnp.arange(U)
    len_mask = (idx_range[None, :] < UNK_lengths[:, None]).astype(jnp.float32)
    all_outputs, all_UNK_weights, all_preds = [], [], []
    W16 = None
    for t in range(4):
        word_emb = _gather(emb_table, last_output)
        x = jnp.concatenate([semantics, styles, word_emb, last_UNK_embeds], axis=-1)
        h = jnp.tanh(x @ W_x + h @ W_h + b_h)
        if t == 0:
            logits, raw_argmax, W16 = _project(h, W_out, b_out, cast="dump")
        else:
            logits, raw_argmax, _ = _project(h, W16, b_out, cast="bf16")
        att = jnp.einsum('bd,bud->bu', h @ W_att, UNK_embeds) / jnp.sqrt(float(D))
        valid = (UNK_mask > 0).astype(jnp.float32) * len_mask
        att = jnp.where(valid > 0, att, -1e9)
        gate = jnp.sum(h * w_gate, axis=-1, keepdims=True)
        UNK_weights = jax.nn.softmax(jnp.concatenate([gate, att], axis=-1), axis=-1)
        last_UNK_embeds = jnp.einsum('bu,bud->bd', UNK_weights[:, 1:], UNK_embeds)
        word_preds = raw_argmax.astype(jnp.int32)
        UNK_argmax = jnp.argmax(UNK_weights[:, 1:], axis=-1)
        is_UNK = (UNK_weights[:, 0] < 0.5).astype(jnp.int32)
        UNK_step_indices = jnp.take_along_axis(UNK_word_ids, UNK_argmax[:, None], axis=1)[:, 0].astype(jnp.int32)
        word_preds = word_preds * (1 - is_UNK) + UNK_step_indices * is_UNK
        last_output = word_preds
        UNK_mask = UNK_mask - (is_UNK[:, None] * (UNK_argmax[:, None] == idx_range[None, :]).astype(jnp.int32)).astype(jnp.float32)
        all_outputs.append(logits)
        all_UNK_weights.append(UNK_weights)
        all_preds.append(word_preds)
    return (jnp.stack(all_outputs, axis=1),
            jnp.stack(all_UNK_weights, axis=1),
            jnp.stack(all_preds, axis=1))
